# baseline (speedup 1.0000x reference)
import jax
import jax.numpy as jnp
from jax.experimental import pallas as pl


def _copy_body(x_ref, o_ref):
    o_ref[...] = x_ref[...]


def kernel(xyz, features):
    del features
    B, N, C = xyz.shape
    CHUNK = 2048
    return pl.pallas_call(
        _copy_body,
        grid=(B, N // CHUNK),
        in_specs=[pl.BlockSpec((1, CHUNK, C), lambda i, j: (i, j, 0))],
        out_specs=pl.BlockSpec((1, CHUNK, C), lambda i, j: (i, j, 0)),
        out_shape=jax.ShapeDtypeStruct((B, N, C), xyz.dtype),
    )(xyz)
